# baseline (speedup 1.0000x reference)
import functools
import math

import jax
import jax.numpy as jnp
from jax import lax
from jax.experimental import pallas as pl
from jax.experimental.pallas import tpu as pltpu
from jax.experimental.pallas import tpu_sc as plsc

_DIM = 32
_NGRAPH = 64
_BN_SCALE = 1.0 / math.sqrt(1.0 + 1e-5)
_CH = 128
_NBUF = 10
_LOOK = 5
_PAD = 16


@functools.lru_cache(maxsize=None)
def _make_segsum(n, e, d):
    info = plsc.get_sparse_core_info()
    nc, ns = info.num_cores, info.num_subcores
    nw = nc * ns
    epw = e // nw
    nchunk = -(-epw // _CH)
    nacc = n + _PAD
    rps = (n // ns) // 8 * 8
    tail = n - ns * rps
    rpz = (nacc // ns) // 8 * 8
    tlz = nacc - ns * rpz
    assert tail % 8 == 0 and tlz % 8 == 0

    mesh = plsc.VectorSubcoreMesh(core_axis_name="c", subcore_axis_name="s")

    @functools.partial(
        pl.kernel,
        mesh=mesh,
        out_type=jax.ShapeDtypeStruct((nc, n, d), jnp.float32),
        scratch_types=[
            pltpu.VMEM((nchunk, _CH), jnp.int32),
            pltpu.VMEM((nchunk, _CH), jnp.int32),
            pltpu.VMEM((_NBUF, _CH, d), jnp.float32),
            pltpu.VMEM_SHARED((nacc, d), jnp.float32),
            pltpu.VMEM_SHARED((n, d), jnp.float32),
            pltpu.SemaphoreType.DMA,
            pltpu.SemaphoreType.DMA((_NBUF,)),
            pltpu.SemaphoreType.DMA((_NBUF,)),
        ],
        compiler_params=pltpu.CompilerParams(use_tc_tiling_on_sc=False),
    )
    def segsum(y_hbm, src_hbm, dst_hbm, zeros_hbm, out_hbm, sidx_v, didx_v,
               rows_v, acc_sh, y_sh, isem, gsem, ssem):
        c = lax.axis_index("c")
        s = lax.axis_index("s")
        wid = s * nc + c
        i0 = pltpu.async_copy(src_hbm.at[wid], sidx_v, isem)
        i1 = pltpu.async_copy(dst_hbm.at[wid], didx_v, isem)
        row0 = pl.multiple_of(s * rps, 8)
        rowz = pl.multiple_of(s * rpz, 8)
        pltpu.sync_copy(zeros_hbm.at[pl.ds(rowz, rpz)],
                        acc_sh.at[pl.ds(rowz, rpz)])
        pltpu.sync_copy(y_hbm.at[pl.ds(row0, rps)],
                        y_sh.at[pl.ds(row0, rps)])

        @pl.when(s == ns - 1)
        def _():
            if tlz:
                pltpu.sync_copy(zeros_hbm.at[pl.ds(ns * rpz, tlz)],
                                acc_sh.at[pl.ds(ns * rpz, tlz)])
            if tail:
                pltpu.sync_copy(y_hbm.at[pl.ds(ns * rps, tail)],
                                y_sh.at[pl.ds(ns * rps, tail)])
        i0.wait()
        i1.wait()
        plsc.subcore_barrier()

        def drain(buf, sem):
            pltpu.make_async_copy(y_hbm.at[pl.ds(0, _CH)], buf, sem).wait()

        def fire_gather(j, b):
            pltpu.async_copy(y_sh.at[sidx_v.at[j]], rows_v.at[b],
                             gsem.at[b])

        def fire_scatter(j, b):
            drain(rows_v.at[b], gsem.at[b])
            pltpu.async_copy(rows_v.at[b], acc_sh.at[didx_v.at[j]],
                             ssem.at[b], add=True)

        for b in range(_LOOK):
            fire_gather(b, b)
        for j in range(_LOOK):
            fire_scatter(j, j)
            fire_gather(j + _LOOK, j + _LOOK)

        @pl.loop(_LOOK, nchunk - _LOOK)
        def _(j):
            b = lax.rem(j, _NBUF)
            fire_scatter(j, b)
            jj = j + _LOOK
            bb = lax.rem(jj, _NBUF)
            drain(rows_v.at[bb], ssem.at[bb])
            fire_gather(jj, bb)

        for j in range(nchunk - _LOOK, nchunk):
            fire_scatter(j, j % _NBUF)
        for b in range(_NBUF):
            drain(rows_v.at[b], ssem.at[b])

        plsc.subcore_barrier()
        pltpu.sync_copy(acc_sh.at[pl.ds(row0, rps)],
                        out_hbm.at[c, pl.ds(row0, rps)])
        if tail:
            @pl.when(s == ns - 1)
            def _():
                pltpu.sync_copy(acc_sh.at[pl.ds(ns * rps, tail)],
                                out_hbm.at[c, pl.ds(ns * rps, tail)])

    return segsum


def _first_mm(x, w):
    def body(x_ref, w_ref, o_ref):
        o_ref[...] = jnp.dot(x_ref[...], w_ref[...],
                             preferred_element_type=jnp.float32)

    return pl.pallas_call(
        body,
        out_shape=jax.ShapeDtypeStruct((x.shape[0], w.shape[1]), jnp.float32),
    )(x, w)


def _node_update(p_ref, y_ref, ba_ref, wb_ref, bb_ref, g_ref, bt_ref):
    z = jnp.maximum(p_ref[0] + p_ref[1] + y_ref[...] + ba_ref[...], 0.0)
    z = jnp.dot(z, wb_ref[...], preferred_element_type=jnp.float32) + bb_ref[...]
    h = jnp.maximum(z, 0.0)
    return h * (g_ref[...] * _BN_SCALE) + bt_ref[...]


def _mid(parts, y, ba, wb, bb, g, bt, wan):
    def body(p_ref, y_ref, ba_ref, wb_ref, bb_ref, g_ref, bt_ref, wan_ref,
             o_ref):
        h = _node_update(p_ref, y_ref, ba_ref, wb_ref, bb_ref, g_ref, bt_ref)
        o_ref[...] = jnp.dot(h, wan_ref[...],
                             preferred_element_type=jnp.float32)

    return pl.pallas_call(
        body,
        out_shape=jax.ShapeDtypeStruct(y.shape, jnp.float32),
    )(parts, y, ba, wb, bb, g, bt, wan)


def _last(parts, y, ba, wb, bb, g, bt, batch2d, fw1, fb1, fw2, fb2):
    n = y.shape[0]
    ncls = fw2.shape[1]

    def body(p_ref, y_ref, ba_ref, wb_ref, bb_ref, g_ref, bt_ref, b_ref,
             fw1_ref, fb1_ref, fw2_ref, fb2_ref, o_ref):
        h = _node_update(p_ref, y_ref, ba_ref, wb_ref, bb_ref, g_ref, bt_ref)
        ids = lax.broadcasted_iota(jnp.int32, (_NGRAPH, n), 0)
        onehot = (ids == b_ref[...]).astype(jnp.float32)
        pooled = jnp.dot(onehot, h, preferred_element_type=jnp.float32)
        t = jnp.maximum(
            jnp.dot(pooled, fw1_ref[...], preferred_element_type=jnp.float32)
            + fb1_ref[...], 0.0)
        oo = (jnp.dot(t, fw2_ref[...], preferred_element_type=jnp.float32)
              + fb2_ref[...])
        m = jnp.max(oo, axis=-1, keepdims=True)
        ls = m + jnp.log(jnp.sum(jnp.exp(oo - m), axis=-1, keepdims=True))
        o_ref[...] = oo - ls

    return pl.pallas_call(
        body,
        out_shape=jax.ShapeDtypeStruct((_NGRAPH, ncls), jnp.float32),
    )(parts, y, ba, wb, bb, g, bt, batch2d, fw1, fb1, fw2, fb2)


def kernel(x, params, edge_index, batch):
    n = x.shape[0]
    zeros = jnp.zeros((n + _PAD, _DIM), jnp.float32)
    edges = edge_index.astype(jnp.int32)
    e = edges.shape[1]
    info = plsc.get_sparse_core_info()
    nw = info.num_cores * info.num_subcores
    epw = e // nw
    nchunk = -(-epw // _CH)
    pad = nchunk * _CH - epw
    src = jnp.pad(edges[0].reshape(nw, epw), ((0, 0), (0, pad)))
    dst = jnp.pad(edges[1].reshape(nw, epw), ((0, 0), (0, pad)),
                  constant_values=n)
    src = src.reshape(nw, nchunk, _CH)
    dst = dst.reshape(nw, nchunk, _CH)
    segsum = _make_segsum(n, e, _DIM)
    b2 = lambda v: v.reshape(1, -1)

    y = _first_mm(x, params["w0a"])
    for i in range(4):
        parts = segsum(y, src, dst, zeros)
        y = _mid(parts, y, b2(params[f"b{i}a"]), params[f"w{i}b"],
                 b2(params[f"b{i}b"]), b2(params[f"g{i}"]),
                 b2(params[f"bt{i}"]), params[f"w{i + 1}a"])
    parts = segsum(y, src, dst, zeros)
    return _last(parts, y, b2(params["b4a"]), params["w4b"],
                 b2(params["b4b"]), b2(params["g4"]), b2(params["bt4"]),
                 batch.astype(jnp.int32).reshape(1, -1),
                 params["fw1"], b2(params["fb1"]), params["fw2"],
                 b2(params["fb2"]))


# speedup vs baseline: 1.0414x; 1.0414x over previous
import functools
import math

import jax
import jax.numpy as jnp
from jax import lax
from jax.experimental import pallas as pl
from jax.experimental.pallas import tpu as pltpu
from jax.experimental.pallas import tpu_sc as plsc

_DIM = 32
_NGRAPH = 64
_BN_SCALE = 1.0 / math.sqrt(1.0 + 1e-5)
_CH = 80
_NBUF = 14
_LOOK = 7
_PAD = 16


@functools.lru_cache(maxsize=None)
def _make_segsum(n, e, d):
    info = plsc.get_sparse_core_info()
    nc, ns = info.num_cores, info.num_subcores
    nw = nc * ns
    epw = e // nw
    nchunk = -(-epw // _CH)
    nacc = n + _PAD
    rps = (n // ns) // 8 * 8
    tail = n - ns * rps
    rpz = (nacc // ns) // 8 * 8
    tlz = nacc - ns * rpz
    assert tail % 8 == 0 and tlz % 8 == 0

    mesh = plsc.VectorSubcoreMesh(core_axis_name="c", subcore_axis_name="s")

    @functools.partial(
        pl.kernel,
        mesh=mesh,
        out_type=jax.ShapeDtypeStruct((nc, n, d), jnp.float32),
        scratch_types=[
            pltpu.VMEM((nchunk, _CH), jnp.int32),
            pltpu.VMEM((nchunk, _CH), jnp.int32),
            pltpu.VMEM((_NBUF, _CH, d), jnp.float32),
            pltpu.VMEM_SHARED((nacc, d), jnp.float32),
            pltpu.VMEM_SHARED((n, d), jnp.float32),
            pltpu.SemaphoreType.DMA,
            pltpu.SemaphoreType.DMA((_NBUF,)),
            pltpu.SemaphoreType.DMA((_NBUF,)),
        ],
        compiler_params=pltpu.CompilerParams(use_tc_tiling_on_sc=False),
    )
    def segsum(y_hbm, src_hbm, dst_hbm, zeros_hbm, out_hbm, sidx_v, didx_v,
               rows_v, acc_sh, y_sh, isem, gsem, ssem):
        c = lax.axis_index("c")
        s = lax.axis_index("s")
        wid = s * nc + c
        i0 = pltpu.async_copy(src_hbm.at[wid], sidx_v, isem)
        i1 = pltpu.async_copy(dst_hbm.at[wid], didx_v, isem)
        row0 = pl.multiple_of(s * rps, 8)
        rowz = pl.multiple_of(s * rpz, 8)
        pltpu.sync_copy(zeros_hbm.at[pl.ds(rowz, rpz)],
                        acc_sh.at[pl.ds(rowz, rpz)])
        pltpu.sync_copy(y_hbm.at[pl.ds(row0, rps)],
                        y_sh.at[pl.ds(row0, rps)])

        @pl.when(s == ns - 1)
        def _():
            if tlz:
                pltpu.sync_copy(zeros_hbm.at[pl.ds(ns * rpz, tlz)],
                                acc_sh.at[pl.ds(ns * rpz, tlz)])
            if tail:
                pltpu.sync_copy(y_hbm.at[pl.ds(ns * rps, tail)],
                                y_sh.at[pl.ds(ns * rps, tail)])
        i0.wait()
        i1.wait()
        plsc.subcore_barrier()

        def drain(buf, sem):
            pltpu.make_async_copy(y_hbm.at[pl.ds(0, _CH)], buf, sem).wait()

        def fire_gather(j, b):
            pltpu.async_copy(y_sh.at[sidx_v.at[j]], rows_v.at[b],
                             gsem.at[b])

        def fire_scatter(j, b):
            drain(rows_v.at[b], gsem.at[b])
            pltpu.async_copy(rows_v.at[b], acc_sh.at[didx_v.at[j]],
                             ssem.at[b], add=True)

        for b in range(_LOOK):
            fire_gather(b, b)
        for j in range(_LOOK):
            fire_scatter(j, j)
            fire_gather(j + _LOOK, j + _LOOK)

        @pl.loop(_LOOK, nchunk - _LOOK)
        def _(j):
            b = lax.rem(j, _NBUF)
            fire_scatter(j, b)
            jj = j + _LOOK
            bb = lax.rem(jj, _NBUF)
            drain(rows_v.at[bb], ssem.at[bb])
            fire_gather(jj, bb)

        for j in range(nchunk - _LOOK, nchunk):
            fire_scatter(j, j % _NBUF)
        for b in range(_NBUF):
            drain(rows_v.at[b], ssem.at[b])

        plsc.subcore_barrier()
        pltpu.sync_copy(acc_sh.at[pl.ds(row0, rps)],
                        out_hbm.at[c, pl.ds(row0, rps)])
        if tail:
            @pl.when(s == ns - 1)
            def _():
                pltpu.sync_copy(acc_sh.at[pl.ds(ns * rps, tail)],
                                out_hbm.at[c, pl.ds(ns * rps, tail)])

    return segsum


def _first_mm(x, w):
    def body(x_ref, w_ref, o_ref):
        o_ref[...] = jnp.dot(x_ref[...], w_ref[...],
                             preferred_element_type=jnp.float32)

    return pl.pallas_call(
        body,
        out_shape=jax.ShapeDtypeStruct((x.shape[0], w.shape[1]), jnp.float32),
    )(x, w)


def _node_update(p_ref, y_ref, ba_ref, wb_ref, bb_ref, g_ref, bt_ref):
    z = jnp.maximum(p_ref[0] + p_ref[1] + y_ref[...] + ba_ref[...], 0.0)
    z = jnp.dot(z, wb_ref[...], preferred_element_type=jnp.float32) + bb_ref[...]
    h = jnp.maximum(z, 0.0)
    return h * (g_ref[...] * _BN_SCALE) + bt_ref[...]


def _mid(parts, y, ba, wb, bb, g, bt, wan):
    def body(p_ref, y_ref, ba_ref, wb_ref, bb_ref, g_ref, bt_ref, wan_ref,
             o_ref):
        h = _node_update(p_ref, y_ref, ba_ref, wb_ref, bb_ref, g_ref, bt_ref)
        o_ref[...] = jnp.dot(h, wan_ref[...],
                             preferred_element_type=jnp.float32)

    return pl.pallas_call(
        body,
        out_shape=jax.ShapeDtypeStruct(y.shape, jnp.float32),
    )(parts, y, ba, wb, bb, g, bt, wan)


def _last(parts, y, ba, wb, bb, g, bt, batch2d, fw1, fb1, fw2, fb2):
    n = y.shape[0]
    ncls = fw2.shape[1]

    def body(p_ref, y_ref, ba_ref, wb_ref, bb_ref, g_ref, bt_ref, b_ref,
             fw1_ref, fb1_ref, fw2_ref, fb2_ref, o_ref):
        h = _node_update(p_ref, y_ref, ba_ref, wb_ref, bb_ref, g_ref, bt_ref)
        ids = lax.broadcasted_iota(jnp.int32, (_NGRAPH, n), 0)
        onehot = (ids == b_ref[...]).astype(jnp.float32)
        pooled = jnp.dot(onehot, h, preferred_element_type=jnp.float32)
        t = jnp.maximum(
            jnp.dot(pooled, fw1_ref[...], preferred_element_type=jnp.float32)
            + fb1_ref[...], 0.0)
        oo = (jnp.dot(t, fw2_ref[...], preferred_element_type=jnp.float32)
              + fb2_ref[...])
        m = jnp.max(oo, axis=-1, keepdims=True)
        ls = m + jnp.log(jnp.sum(jnp.exp(oo - m), axis=-1, keepdims=True))
        o_ref[...] = oo - ls

    return pl.pallas_call(
        body,
        out_shape=jax.ShapeDtypeStruct((_NGRAPH, ncls), jnp.float32),
    )(parts, y, ba, wb, bb, g, bt, batch2d, fw1, fb1, fw2, fb2)


def kernel(x, params, edge_index, batch):
    n = x.shape[0]
    zeros = jnp.zeros((n + _PAD, _DIM), jnp.float32)
    edges = edge_index.astype(jnp.int32)
    e = edges.shape[1]
    info = plsc.get_sparse_core_info()
    nw = info.num_cores * info.num_subcores
    epw = e // nw
    nchunk = -(-epw // _CH)
    pad = nchunk * _CH - epw
    src = jnp.pad(edges[0].reshape(nw, epw), ((0, 0), (0, pad)))
    dst = jnp.pad(edges[1].reshape(nw, epw), ((0, 0), (0, pad)),
                  constant_values=n)
    src = src.reshape(nw, nchunk, _CH)
    dst = dst.reshape(nw, nchunk, _CH)
    segsum = _make_segsum(n, e, _DIM)
    b2 = lambda v: v.reshape(1, -1)

    y = _first_mm(x, params["w0a"])
    for i in range(4):
        parts = segsum(y, src, dst, zeros)
        y = _mid(parts, y, b2(params[f"b{i}a"]), params[f"w{i}b"],
                 b2(params[f"b{i}b"]), b2(params[f"g{i}"]),
                 b2(params[f"bt{i}"]), params[f"w{i + 1}a"])
    parts = segsum(y, src, dst, zeros)
    return _last(parts, y, b2(params["b4a"]), params["w4b"],
                 b2(params["b4b"]), b2(params["g4"]), b2(params["bt4"]),
                 batch.astype(jnp.int32).reshape(1, -1),
                 params["fw1"], b2(params["fb1"]), params["fw2"],
                 b2(params["fb2"]))
